# Initial kernel scaffold; baseline (speedup 1.0000x reference)
import jax
import jax.numpy as jnp
from jax.experimental import pallas as pl


def kernel(yhat, y):
    raise NotImplementedError("write your pallas kernel here")


# speedup vs baseline: 111.2069x; 111.2069x over previous
import functools

import jax
import jax.numpy as jnp
from jax import lax
from jax.experimental import pallas as pl
from jax.experimental.pallas import tpu as pltpu
from jax.experimental.pallas import tpu_sc as plsc

B = 262144
C = 128

R = 2048
NB = B // R


def _argmax_body(x_ref, o_ref):
    x = x_ref[...]
    m = jnp.max(x, axis=1, keepdims=True)
    ii = lax.broadcasted_iota(jnp.int32, (R, C), 1)
    idx = jnp.min(jnp.where(x == m, ii, C), axis=1).astype(jnp.int32)
    o_ref[0, 0, :] = idx


def _argmax_call(yhat):
    return pl.pallas_call(
        _argmax_body,
        grid=(NB,),
        in_specs=[pl.BlockSpec((R, C), lambda i: (i, jnp.int32(0)))],
        out_specs=pl.BlockSpec((1, 1, R),
                               lambda i: (i, jnp.int32(0), jnp.int32(0))),
        out_shape=jax.ShapeDtypeStruct((NB, 1, R), jnp.int32),
    )(yhat)


NC, NS, L = 2, 16, 16
NW = NC * NS
CHUNK = B // NW
NIT = CHUNK // L
CV = C // L

_SC_PARAMS = dict(
    compiler_params=pltpu.CompilerParams(needs_layout_passes=False),
)


def _wid():
    return lax.axis_index("s") * jnp.int32(NC) + lax.axis_index("c")


@functools.cache
def _build_sc_presence():
    mesh = plsc.VectorSubcoreMesh(core_axis_name="c", subcore_axis_name="s")
    return functools.partial(
        pl.kernel,
        out_type=jax.ShapeDtypeStruct((NW, C), jnp.int32),
        mesh=mesh,
        scratch_types=[
            pltpu.VMEM((CHUNK,), jnp.int32),
            pltpu.VMEM((C,), jnp.int32),
        ],
        **_SC_PARAMS,
    )(_sc_presence_body)


def _sc_presence_body(y_hbm, pres_out, y_v, pres_v):
    wid = _wid()
    base = wid * jnp.int32(CHUNK)
    pltpu.sync_copy(y_hbm.at[pl.ds(base, CHUNK)], y_v)

    zeros = jnp.zeros((L,), jnp.int32)
    ones = jnp.ones((L,), jnp.int32)
    for j in range(CV):
        pres_v[pl.ds(j * L, L)] = zeros

    def body(k, carry):
        yv = y_v[pl.ds(k * jnp.int32(L), L)]
        plsc.store_scatter(pres_v, [yv], ones)
        return carry

    lax.fori_loop(jnp.int32(0), jnp.int32(NIT), body, jnp.int32(0))
    pltpu.sync_copy(pres_v, pres_out.at[wid])


@functools.cache
def _build_sc_match():
    mesh = plsc.VectorSubcoreMesh(core_axis_name="c", subcore_axis_name="s")
    return functools.partial(
        pl.kernel,
        out_type=jax.ShapeDtypeStruct((NW, L), jnp.float32),
        mesh=mesh,
        scratch_types=[
            pltpu.VMEM((CHUNK,), jnp.int32),
            pltpu.VMEM((CHUNK,), jnp.int32),
            pltpu.VMEM((NW, C), jnp.int32),
            pltpu.VMEM((C,), jnp.int32),
            pltpu.VMEM((L,), jnp.float32),
        ],
        **_SC_PARAMS,
    )(_sc_match_body)


def _sc_match_body(y_hbm, idx_hbm, pres_hbm, part_out,
                   y_v, idx_v, allpres_v, table_v, stage_v):
    wid = _wid()
    base = wid * jnp.int32(CHUNK)
    iota = lax.iota(jnp.int32, L)

    pltpu.sync_copy(y_hbm.at[pl.ds(base, CHUNK)], y_v)
    pltpu.sync_copy(idx_hbm.at[pl.ds(base, CHUNK)], idx_v)
    pltpu.sync_copy(pres_hbm, allpres_v)

    pres_vecs = []
    for j in range(CV):
        a = allpres_v[0, pl.ds(j * L, L)]
        for t in range(1, NW):
            a = a | allpres_v[t, pl.ds(j * L, L)]
        pres_vecs.append(a > 0)

    maxv = jnp.int32(-1)
    for j in range(CV):
        vals = iota + j * L
        maxv = jnp.maximum(maxv, jnp.max(jnp.where(pres_vecs[j], vals, -1)))

    for j in range(CV):
        table_v[pl.ds(j * L, L)] = jnp.broadcast_to(maxv, (L,))
    carry = jnp.int32(0)
    for j in range(CV):
        p32 = pres_vecs[j].astype(jnp.int32)
        rank = plsc.cumsum(p32) + carry - 1
        carry = carry + jnp.sum(p32, dtype=jnp.int32)
        vals = iota + j * L
        plsc.store_scatter(table_v, [rank], vals, mask=pres_vecs[j])

    def match_body(k, acc):
        off = k * jnp.int32(L)
        yv = y_v[pl.ds(off, L)]
        jv = idx_v[pl.ds(off, L)]
        g = plsc.load_gather(table_v, [jv])
        gi = base + off + iota
        return acc + jnp.where(g == yv, gi, 0)

    acc = lax.fori_loop(jnp.int32(0), jnp.int32(NIT), match_body,
                        jnp.zeros((L,), jnp.int32))
    stage_v[...] = acc.astype(jnp.float32)
    pltpu.sync_copy(stage_v, part_out.at[wid])


def _finish_body(p_ref, o_ref):
    loss = jnp.float32(1.0) - jnp.sum(p_ref[...]) * jnp.float32(1.0 / B)
    o_ref[...] = jnp.broadcast_to(loss, (1, 1))


def _finish_call(parts):
    return pl.pallas_call(
        _finish_body,
        out_shape=jax.ShapeDtypeStruct((1, 1), jnp.float32),
    )(parts)


def kernel(yhat, y):
    y32 = y.astype(jnp.int32)
    pres = _build_sc_presence()(y32)
    idx = _argmax_call(yhat).reshape(B)
    parts = _build_sc_match()(y32, idx, pres)
    loss = _finish_call(parts.reshape(NW * L // C, C))
    return loss[0, 0]


# speedup vs baseline: 143.0605x; 1.2864x over previous
import functools

import jax
import jax.numpy as jnp
from jax import lax
from jax.experimental import pallas as pl
from jax.experimental.pallas import tpu as pltpu
from jax.experimental.pallas import tpu_sc as plsc

B = 262144
C = 128

R = 2048
NB = B // R

NC, NS, L = 2, 16, 16
NW = NC * NS
CHUNK = B // NW
NIT = CHUNK // L
CV = C // L

_SC_PARAMS = dict(
    compiler_params=pltpu.CompilerParams(needs_layout_passes=False),
)


def _wid():
    return lax.axis_index("s") * jnp.int32(NC) + lax.axis_index("c")


@functools.cache
def _build_sc_presence():
    mesh = plsc.VectorSubcoreMesh(core_axis_name="c", subcore_axis_name="s")
    return functools.partial(
        pl.kernel,
        out_type=jax.ShapeDtypeStruct((NW, C), jnp.int32),
        mesh=mesh,
        scratch_types=[
            pltpu.VMEM((CHUNK,), jnp.int32),
            pltpu.VMEM((C,), jnp.int32),
        ],
        **_SC_PARAMS,
    )(_sc_presence_body)


def _sc_presence_body(y_hbm, pres_out, y_v, pres_v):
    wid = _wid()
    base = wid * jnp.int32(CHUNK)
    pltpu.sync_copy(y_hbm.at[pl.ds(base, CHUNK)], y_v)

    zeros = jnp.zeros((L,), jnp.int32)
    ones = jnp.ones((L,), jnp.int32)
    for j in range(CV):
        pres_v[pl.ds(j * L, L)] = zeros

    def body(k, carry):
        yv = y_v[pl.ds(k * jnp.int32(L), L)]
        plsc.store_scatter(pres_v, [yv], ones)
        return carry

    lax.fori_loop(jnp.int32(0), jnp.int32(NIT), body, jnp.int32(0))
    pltpu.sync_copy(pres_v, pres_out.at[wid])


@functools.cache
def _build_sc_table():
    mesh = plsc.VectorSubcoreMesh(core_axis_name="c", subcore_axis_name="s")
    return functools.partial(
        pl.kernel,
        out_type=jax.ShapeDtypeStruct((C,), jnp.int32),
        mesh=mesh,
        scratch_types=[
            pltpu.VMEM((NW, C), jnp.int32),
            pltpu.VMEM((C,), jnp.int32),
        ],
        **_SC_PARAMS,
    )(_sc_table_body)


def _sc_table_body(pres_hbm, tab_out, allpres_v, table_v):
    wid = _wid()
    iota = lax.iota(jnp.int32, L)
    pltpu.sync_copy(pres_hbm, allpres_v)

    pres_vecs = []
    for j in range(CV):
        a = allpres_v[0, pl.ds(j * L, L)]
        for t in range(1, NW):
            a = a | allpres_v[t, pl.ds(j * L, L)]
        pres_vecs.append(a > 0)

    maxv = jnp.int32(-1)
    for j in range(CV):
        vals = iota + j * L
        maxv = jnp.maximum(maxv, jnp.max(jnp.where(pres_vecs[j], vals, -1)))

    for j in range(CV):
        table_v[pl.ds(j * L, L)] = jnp.broadcast_to(maxv, (L,))
    carry = jnp.int32(0)
    for j in range(CV):
        p32 = pres_vecs[j].astype(jnp.int32)
        rank = plsc.cumsum(p32) + carry - 1
        carry = carry + jnp.sum(p32, dtype=jnp.int32)
        vals = iota + j * L
        plsc.store_scatter(table_v, [rank], vals, mask=pres_vecs[j])

    @pl.when(wid == 0)
    def _():
        pltpu.sync_copy(table_v, tab_out)


def _sweep_body(x_ref, y_ref, ii_ref, vi_ref, w_ref, o_ref):
    b = pl.program_id(0)
    x = x_ref[...]
    rowmax = jnp.max(x, axis=1, keepdims=True)
    ii = ii_ref[...]
    minc = jnp.min(jnp.where(x == rowmax, ii, jnp.float32(C)),
                   axis=1, keepdims=True)
    fb = (ii == minc).astype(jnp.bfloat16)

    yb = y_ref[0, 0, :].astype(jnp.bfloat16)
    yoht = (vi_ref[...] == yb[None, :]).astype(jnp.bfloat16)

    yoht0 = yoht * w_ref[0:1, :]
    yohtq = yoht * w_ref[1:2, :]
    lhs = jnp.concatenate([yoht, yohtq, yoht0], axis=0)
    s = jnp.dot(lhs, fb, preferred_element_type=jnp.float32)

    bf = b.astype(jnp.float32)

    @pl.when(b == 0)
    def _():
        o_ref[0:C, :] = jnp.zeros((C, C), jnp.float32)
        o_ref[C:2 * C, :] = s[C:2 * C, :]
        o_ref[2 * C:3 * C, :] = s[2 * C:3 * C, :]

    @pl.when(b != 0)
    def _():
        o_ref[0:C, :] += bf * s[0:C, :]
        o_ref[C:2 * C, :] += s[C:2 * C, :]
        o_ref[2 * C:3 * C, :] += s[2 * C:3 * C, :]


def _sweep_call(yhat, y3d):
    ii = lax.broadcasted_iota(jnp.int32, (R, C), 1).astype(jnp.float32)
    vi = lax.broadcasted_iota(jnp.int32, (C, R), 0).astype(jnp.bfloat16)
    rr = lax.broadcasted_iota(jnp.int32, (8, R), 1)
    w = jnp.where(lax.broadcasted_iota(jnp.int32, (8, R), 0) == 0,
                  rr & 127, rr >> 7).astype(jnp.bfloat16)
    z = lambda: jnp.int32(0)
    return pl.pallas_call(
        _sweep_body,
        grid=(NB,),
        in_specs=[
            pl.BlockSpec((R, C), lambda i: (i, z())),
            pl.BlockSpec((1, 1, R), lambda i: (i, z(), z())),
            pl.BlockSpec((R, C), lambda i: (z(), z())),
            pl.BlockSpec((C, R), lambda i: (z(), z())),
            pl.BlockSpec((8, R), lambda i: (z(), z())),
        ],
        out_specs=pl.BlockSpec((3 * C, C), lambda i: (z(), z())),
        out_shape=jax.ShapeDtypeStruct((3 * C, C), jnp.float32),
    )(yhat, y3d, ii, vi, w)


def _fold_body(a_ref, t_ref, o_ref):
    tbl = t_ref[0, :]
    vi = lax.broadcasted_iota(jnp.int32, (C, C), 0)
    t = (vi == tbl[None, :]).astype(jnp.float32)
    s = (jnp.float32(2048.0) * a_ref[0:C, :]
         + jnp.float32(128.0) * a_ref[C:2 * C, :]
         + a_ref[2 * C:3 * C, :])
    idx_sum = jnp.sum(t * s)
    loss = jnp.float32(1.0) - idx_sum * jnp.float32(1.0 / B)
    o_ref[...] = jnp.broadcast_to(loss, (1, 1))


def _fold_call(accs, table2d):
    return pl.pallas_call(
        _fold_body,
        out_shape=jax.ShapeDtypeStruct((1, 1), jnp.float32),
    )(accs, table2d)


def kernel(yhat, y):
    y32 = y.astype(jnp.int32)
    pres = _build_sc_presence()(y32)
    table = _build_sc_table()(pres)
    accs = _sweep_call(yhat, y32.reshape(NB, 1, R))
    loss = _fold_call(accs, table.reshape(1, C))
    return loss[0, 0]


# speedup vs baseline: 179.2158x; 1.2527x over previous
import functools

import jax
import jax.numpy as jnp
from jax import lax
from jax.experimental import pallas as pl
from jax.experimental.pallas import tpu as pltpu
from jax.experimental.pallas import tpu_sc as plsc

B = 262144
C = 128

R = 8192
NB = B // R

NC, NS, L = 2, 16, 16
NW = NC * NS
CHUNK = B // NW
NIT = CHUNK // L
CV = C // L

_SC_PARAMS = dict(
    compiler_params=pltpu.CompilerParams(needs_layout_passes=False),
)


def _wid():
    return lax.axis_index("s") * jnp.int32(NC) + lax.axis_index("c")


@functools.cache
def _build_sc_presence():
    mesh = plsc.VectorSubcoreMesh(core_axis_name="c", subcore_axis_name="s")
    return functools.partial(
        pl.kernel,
        out_type=jax.ShapeDtypeStruct((NW, C), jnp.int32),
        mesh=mesh,
        scratch_types=[
            pltpu.VMEM((CHUNK,), jnp.int32),
            pltpu.VMEM((C,), jnp.int32),
        ],
        **_SC_PARAMS,
    )(_sc_presence_body)


def _sc_presence_body(y_hbm, pres_out, y_v, pres_v):
    wid = _wid()
    base = wid * jnp.int32(CHUNK)
    pltpu.sync_copy(y_hbm.at[pl.ds(base, CHUNK)], y_v)

    zeros = jnp.zeros((L,), jnp.int32)
    ones = jnp.ones((L,), jnp.int32)
    for j in range(CV):
        pres_v[pl.ds(j * L, L)] = zeros

    def body(k, carry):
        yv = y_v[pl.ds(k * jnp.int32(L), L)]
        plsc.store_scatter(pres_v, [yv], ones)
        return carry

    lax.fori_loop(jnp.int32(0), jnp.int32(NIT), body, jnp.int32(0))
    pltpu.sync_copy(pres_v, pres_out.at[wid])


@functools.cache
def _build_sc_table():
    mesh = plsc.VectorSubcoreMesh(core_axis_name="c", subcore_axis_name="s")
    return functools.partial(
        pl.kernel,
        out_type=jax.ShapeDtypeStruct((C,), jnp.int32),
        mesh=mesh,
        scratch_types=[
            pltpu.VMEM((NW, C), jnp.int32),
            pltpu.VMEM((C,), jnp.int32),
        ],
        **_SC_PARAMS,
    )(_sc_table_body)


def _sc_table_body(pres_hbm, tab_out, allpres_v, table_v):
    wid = _wid()
    iota = lax.iota(jnp.int32, L)
    pltpu.sync_copy(pres_hbm, allpres_v)

    pres_vecs = []
    for j in range(CV):
        a = allpres_v[0, pl.ds(j * L, L)]
        for t in range(1, NW):
            a = a | allpres_v[t, pl.ds(j * L, L)]
        pres_vecs.append(a > 0)

    maxv = jnp.int32(-1)
    for j in range(CV):
        vals = iota + j * L
        maxv = jnp.maximum(maxv, jnp.max(jnp.where(pres_vecs[j], vals, -1)))

    for j in range(CV):
        table_v[pl.ds(j * L, L)] = jnp.broadcast_to(maxv, (L,))
    carry = jnp.int32(0)
    for j in range(CV):
        p32 = pres_vecs[j].astype(jnp.int32)
        rank = plsc.cumsum(p32) + carry - 1
        carry = carry + jnp.sum(p32, dtype=jnp.int32)
        vals = iota + j * L
        plsc.store_scatter(table_v, [rank], vals, mask=pres_vecs[j])

    @pl.when(wid == 0)
    def _():
        pltpu.sync_copy(table_v, tab_out)


def _sweep_body(x_ref, y_ref, ii_ref, vi_ref, w_ref, t_ref, o_ref, acc_ref):
    b = pl.program_id(0)
    x = x_ref[...]
    rowmax = jnp.max(x, axis=1, keepdims=True)
    ii = ii_ref[...]
    minc = jnp.min(jnp.where(x == rowmax, ii, jnp.float32(C)),
                   axis=1, keepdims=True)
    fb = (ii == minc).astype(jnp.bfloat16)

    yb = y_ref[0, 0, :].astype(jnp.bfloat16)
    yoht = (vi_ref[...] == yb[None, :]).astype(jnp.bfloat16)

    yoht0 = yoht * w_ref[0:1, :]
    yohtq = yoht * w_ref[1:2, :]
    lhs = jnp.concatenate([yoht, yohtq, yoht0], axis=0)
    s = jnp.dot(lhs, fb, preferred_element_type=jnp.float32)

    bf = b.astype(jnp.float32)

    @pl.when(b == 0)
    def _():
        acc_ref[0:C, :] = jnp.zeros((C, C), jnp.float32)
        acc_ref[C:2 * C, :] = s[C:2 * C, :]
        acc_ref[2 * C:3 * C, :] = s[2 * C:3 * C, :]

    @pl.when(b != 0)
    def _():
        acc_ref[0:C, :] += bf * s[0:C, :]
        acc_ref[C:2 * C, :] += s[C:2 * C, :]
        acc_ref[2 * C:3 * C, :] += s[2 * C:3 * C, :]

    @pl.when(b == NB - 1)
    def _():
        tbl = t_ref[0, :]
        vv = lax.broadcasted_iota(jnp.int32, (C, C), 0)
        t = (vv == tbl[None, :]).astype(jnp.float32)
        sm = (jnp.float32(float(R)) * acc_ref[0:C, :]
              + jnp.float32(128.0) * acc_ref[C:2 * C, :]
              + acc_ref[2 * C:3 * C, :])
        idx_sum = jnp.sum(t * sm)
        loss = jnp.float32(1.0) - idx_sum * jnp.float32(1.0 / B)
        o_ref[...] = jnp.broadcast_to(loss, (1, 1))


def _sweep_call(yhat, y3d, table2d):
    ii = lax.broadcasted_iota(jnp.int32, (R, C), 1).astype(jnp.float32)
    vi = lax.broadcasted_iota(jnp.int32, (C, R), 0).astype(jnp.bfloat16)
    rr = lax.broadcasted_iota(jnp.int32, (8, R), 1)
    w = jnp.where(lax.broadcasted_iota(jnp.int32, (8, R), 0) == 0,
                  rr & 127, rr >> 7).astype(jnp.bfloat16)
    z = lambda: jnp.int32(0)
    return pl.pallas_call(
        _sweep_body,
        grid=(NB,),
        in_specs=[
            pl.BlockSpec((R, C), lambda i: (i, z())),
            pl.BlockSpec((1, 1, R), lambda i: (i, z(), z())),
            pl.BlockSpec((R, C), lambda i: (z(), z())),
            pl.BlockSpec((C, R), lambda i: (z(), z())),
            pl.BlockSpec((8, R), lambda i: (z(), z())),
            pl.BlockSpec((1, C), lambda i: (z(), z())),
        ],
        out_specs=pl.BlockSpec((1, 1), lambda i: (z(), z())),
        out_shape=jax.ShapeDtypeStruct((1, 1), jnp.float32),
        scratch_shapes=[pltpu.VMEM((3 * C, C), jnp.float32)],
    )(yhat, y3d, ii, vi, w, table2d)


def kernel(yhat, y):
    y32 = y.astype(jnp.int32)
    pres = _build_sc_presence()(y32)
    table = _build_sc_table()(pres)
    loss = _sweep_call(yhat, y32.reshape(NB, 1, R), table.reshape(1, C))
    return loss[0, 0]


# speedup vs baseline: 182.7100x; 1.0195x over previous
import functools

import jax
import jax.numpy as jnp
from jax import lax
from jax.experimental import pallas as pl
from jax.experimental.pallas import tpu as pltpu
from jax.experimental.pallas import tpu_sc as plsc

B = 262144
C = 128

R = 8192
NB = B // R

NC, NS, L = 2, 16, 16
NW = NC * NS
CHUNK = B // NW
NIT = CHUNK // L
CV = C // L

_SC_PARAMS = dict(
    compiler_params=pltpu.CompilerParams(needs_layout_passes=False),
)


def _wid():
    return lax.axis_index("s") * jnp.int32(NC) + lax.axis_index("c")


@functools.cache
def _build_sc_presence():
    mesh = plsc.VectorSubcoreMesh(core_axis_name="c", subcore_axis_name="s")
    return functools.partial(
        pl.kernel,
        out_type=jax.ShapeDtypeStruct((NW, C), jnp.int32),
        mesh=mesh,
        scratch_types=[
            pltpu.VMEM((CHUNK,), jnp.int32),
            pltpu.VMEM((C,), jnp.int32),
        ],
        **_SC_PARAMS,
    )(_sc_presence_body)


def _sc_presence_body(y_hbm, pres_out, y_v, pres_v):
    wid = _wid()
    base = wid * jnp.int32(CHUNK)
    pltpu.sync_copy(y_hbm.at[pl.ds(base, CHUNK)], y_v)

    zeros = jnp.zeros((L,), jnp.int32)
    ones = jnp.ones((L,), jnp.int32)
    for j in range(CV):
        pres_v[pl.ds(j * L, L)] = zeros

    def body(k, carry):
        yv = y_v[pl.ds(k * jnp.int32(L), L)]
        plsc.store_scatter(pres_v, [yv], ones)
        return carry

    lax.fori_loop(jnp.int32(0), jnp.int32(NIT), body, jnp.int32(0))
    pltpu.sync_copy(pres_v, pres_out.at[wid])


@functools.cache
def _build_sc_table():
    mesh = plsc.VectorSubcoreMesh(core_axis_name="c", subcore_axis_name="s")
    return functools.partial(
        pl.kernel,
        out_type=jax.ShapeDtypeStruct((C,), jnp.int32),
        mesh=mesh,
        scratch_types=[
            pltpu.VMEM((NW, C), jnp.int32),
            pltpu.VMEM((C,), jnp.int32),
        ],
        **_SC_PARAMS,
    )(_sc_table_body)


def _sc_table_body(pres_hbm, tab_out, allpres_v, table_v):
    wid = _wid()
    iota = lax.iota(jnp.int32, L)
    pltpu.sync_copy(pres_hbm, allpres_v)

    pres_vecs = []
    for j in range(CV):
        a = allpres_v[0, pl.ds(j * L, L)]
        for t in range(1, NW):
            a = a | allpres_v[t, pl.ds(j * L, L)]
        pres_vecs.append(a > 0)

    maxv = jnp.int32(-1)
    for j in range(CV):
        vals = iota + j * L
        maxv = jnp.maximum(maxv, jnp.max(jnp.where(pres_vecs[j], vals, -1)))

    for j in range(CV):
        table_v[pl.ds(j * L, L)] = jnp.broadcast_to(maxv, (L,))
    carry = jnp.int32(0)
    for j in range(CV):
        p32 = pres_vecs[j].astype(jnp.int32)
        rank = plsc.cumsum(p32) + carry - 1
        carry = carry + jnp.sum(p32, dtype=jnp.int32)
        vals = iota + j * L
        plsc.store_scatter(table_v, [rank], vals, mask=pres_vecs[j])

    @pl.when(wid == 0)
    def _():
        pltpu.sync_copy(table_v, tab_out)


def _sweep_body(x_ref, y_ref, ii_ref, vi_ref, w_ref, o_ref):
    b = pl.program_id(0)
    x = x_ref[...]
    rowmax = jnp.max(x, axis=1, keepdims=True)
    ii = ii_ref[...]
    minc = jnp.min(jnp.where(x == rowmax, ii, jnp.float32(C)),
                   axis=1, keepdims=True)
    fb = (ii == minc).astype(jnp.bfloat16)

    yb = y_ref[0, 0, :].astype(jnp.bfloat16)
    yoht = (vi_ref[...] == yb[None, :]).astype(jnp.bfloat16)

    yoht0 = yoht * w_ref[0:1, :]
    yohtq = yoht * w_ref[1:2, :]
    lhs = jnp.concatenate([yoht, yohtq, yoht0], axis=0)
    s = jnp.dot(lhs, fb, preferred_element_type=jnp.float32)

    bf = b.astype(jnp.float32)

    @pl.when(b == 0)
    def _():
        o_ref[0:C, :] = jnp.zeros((C, C), jnp.float32)
        o_ref[C:2 * C, :] = s[C:2 * C, :]
        o_ref[2 * C:3 * C, :] = s[2 * C:3 * C, :]

    @pl.when(b != 0)
    def _():
        o_ref[0:C, :] += bf * s[0:C, :]
        o_ref[C:2 * C, :] += s[C:2 * C, :]
        o_ref[2 * C:3 * C, :] += s[2 * C:3 * C, :]

def _sweep_call(yhat, y3d):
    ii = lax.broadcasted_iota(jnp.int32, (R, C), 1).astype(jnp.float32)
    vi = lax.broadcasted_iota(jnp.int32, (C, R), 0).astype(jnp.bfloat16)
    rr = lax.broadcasted_iota(jnp.int32, (8, R), 1)
    w = jnp.where(lax.broadcasted_iota(jnp.int32, (8, R), 0) == 0,
                  rr & 127, rr >> 7).astype(jnp.bfloat16)
    z = lambda: jnp.int32(0)
    return pl.pallas_call(
        _sweep_body,
        grid=(NB,),
        in_specs=[
            pl.BlockSpec((R, C), lambda i: (i, z())),
            pl.BlockSpec((1, 1, R), lambda i: (i, z(), z())),
            pl.BlockSpec((R, C), lambda i: (z(), z())),
            pl.BlockSpec((C, R), lambda i: (z(), z())),
            pl.BlockSpec((8, R), lambda i: (z(), z())),
        ],
        out_specs=pl.BlockSpec((3 * C, C), lambda i: (z(), z())),
        out_shape=jax.ShapeDtypeStruct((3 * C, C), jnp.float32),
    )(yhat, y3d, ii, vi, w)


def _fold_body(a_ref, t_ref, o_ref):
    tbl = t_ref[0, :]
    vi = lax.broadcasted_iota(jnp.int32, (C, C), 0)
    t = (vi == tbl[None, :]).astype(jnp.float32)
    s = (jnp.float32(float(R)) * a_ref[0:C, :]
         + jnp.float32(128.0) * a_ref[C:2 * C, :]
         + a_ref[2 * C:3 * C, :])
    idx_sum = jnp.sum(t * s)
    loss = jnp.float32(1.0) - idx_sum * jnp.float32(1.0 / B)
    o_ref[...] = jnp.broadcast_to(loss, (1, 1))


def _fold_call(accs, table2d):
    return pl.pallas_call(
        _fold_body,
        out_shape=jax.ShapeDtypeStruct((1, 1), jnp.float32),
    )(accs, table2d)


def kernel(yhat, y):
    y32 = y.astype(jnp.int32)
    pres = _build_sc_presence()(y32)
    table = _build_sc_table()(pres)
    accs = _sweep_call(yhat, y32.reshape(NB, 1, R))
    loss = _fold_call(accs, table.reshape(1, C))
    return loss[0, 0]


# speedup vs baseline: 183.3542x; 1.0035x over previous
import functools

import jax
import jax.numpy as jnp
from jax import lax
from jax.experimental import pallas as pl
from jax.experimental.pallas import tpu as pltpu
from jax.experimental.pallas import tpu_sc as plsc

B = 262144
C = 128

R = 16384
NB = B // R

NC, NS, L = 2, 16, 16
NW = NC * NS
CHUNK = B // NW
NIT = CHUNK // L
CV = C // L

_SC_PARAMS = dict(
    compiler_params=pltpu.CompilerParams(needs_layout_passes=False),
)


def _wid():
    return lax.axis_index("s") * jnp.int32(NC) + lax.axis_index("c")


@functools.cache
def _build_sc_presence():
    mesh = plsc.VectorSubcoreMesh(core_axis_name="c", subcore_axis_name="s")
    return functools.partial(
        pl.kernel,
        out_type=jax.ShapeDtypeStruct((NW, C), jnp.int32),
        mesh=mesh,
        scratch_types=[
            pltpu.VMEM((CHUNK,), jnp.int32),
            pltpu.VMEM((C,), jnp.int32),
        ],
        **_SC_PARAMS,
    )(_sc_presence_body)


def _sc_presence_body(y_hbm, pres_out, y_v, pres_v):
    wid = _wid()
    base = wid * jnp.int32(CHUNK)
    pltpu.sync_copy(y_hbm.at[pl.ds(base, CHUNK)], y_v)

    zeros = jnp.zeros((L,), jnp.int32)
    ones = jnp.ones((L,), jnp.int32)
    for j in range(CV):
        pres_v[pl.ds(j * L, L)] = zeros

    def body(k, carry):
        yv = y_v[pl.ds(k * jnp.int32(L), L)]
        plsc.store_scatter(pres_v, [yv], ones)
        return carry

    lax.fori_loop(jnp.int32(0), jnp.int32(NIT), body, jnp.int32(0))
    pltpu.sync_copy(pres_v, pres_out.at[wid])


@functools.cache
def _build_sc_table():
    mesh = plsc.VectorSubcoreMesh(core_axis_name="c", subcore_axis_name="s")
    return functools.partial(
        pl.kernel,
        out_type=jax.ShapeDtypeStruct((C,), jnp.int32),
        mesh=mesh,
        scratch_types=[
            pltpu.VMEM((NW, C), jnp.int32),
            pltpu.VMEM((C,), jnp.int32),
        ],
        **_SC_PARAMS,
    )(_sc_table_body)


def _sc_table_body(pres_hbm, tab_out, allpres_v, table_v):
    wid = _wid()
    iota = lax.iota(jnp.int32, L)
    pltpu.sync_copy(pres_hbm, allpres_v)

    pres_vecs = []
    for j in range(CV):
        a = allpres_v[0, pl.ds(j * L, L)]
        for t in range(1, NW):
            a = a | allpres_v[t, pl.ds(j * L, L)]
        pres_vecs.append(a > 0)

    maxv = jnp.int32(-1)
    for j in range(CV):
        vals = iota + j * L
        maxv = jnp.maximum(maxv, jnp.max(jnp.where(pres_vecs[j], vals, -1)))

    for j in range(CV):
        table_v[pl.ds(j * L, L)] = jnp.broadcast_to(maxv, (L,))
    carry = jnp.int32(0)
    for j in range(CV):
        p32 = pres_vecs[j].astype(jnp.int32)
        rank = plsc.cumsum(p32) + carry - 1
        carry = carry + jnp.sum(p32, dtype=jnp.int32)
        vals = iota + j * L
        plsc.store_scatter(table_v, [rank], vals, mask=pres_vecs[j])

    @pl.when(wid == 0)
    def _():
        pltpu.sync_copy(table_v, tab_out)


def _sweep_body(x_ref, y_ref, ii_ref, vi_ref, w_ref, o_ref):
    b = pl.program_id(0)
    x = x_ref[...]
    rowmax = jnp.max(x, axis=1, keepdims=True)
    ii = ii_ref[...]
    minc = jnp.min(jnp.where(x == rowmax, ii, jnp.float32(C)),
                   axis=1, keepdims=True)
    fb = (ii == minc).astype(jnp.bfloat16)

    yb = y_ref[0, 0, :].astype(jnp.bfloat16)
    yoht = (vi_ref[...] == yb[None, :]).astype(jnp.bfloat16)

    yoht0 = yoht * w_ref[0:1, :]
    yohtq = yoht * w_ref[1:2, :]
    lhs = jnp.concatenate([yoht, yohtq, yoht0], axis=0)
    s = jnp.dot(lhs, fb, preferred_element_type=jnp.float32)

    bf = b.astype(jnp.float32)

    @pl.when(b == 0)
    def _():
        o_ref[0:C, :] = jnp.zeros((C, C), jnp.float32)
        o_ref[C:2 * C, :] = s[C:2 * C, :]
        o_ref[2 * C:3 * C, :] = s[2 * C:3 * C, :]

    @pl.when(b != 0)
    def _():
        o_ref[0:C, :] += bf * s[0:C, :]
        o_ref[C:2 * C, :] += s[C:2 * C, :]
        o_ref[2 * C:3 * C, :] += s[2 * C:3 * C, :]

def _sweep_call(yhat, y3d):
    ii = lax.broadcasted_iota(jnp.int32, (R, C), 1).astype(jnp.float32)
    vi = lax.broadcasted_iota(jnp.int32, (C, R), 0).astype(jnp.bfloat16)
    rr = lax.broadcasted_iota(jnp.int32, (8, R), 1)
    w = jnp.where(lax.broadcasted_iota(jnp.int32, (8, R), 0) == 0,
                  rr & 127, rr >> 7).astype(jnp.bfloat16)
    z = lambda: jnp.int32(0)
    return pl.pallas_call(
        _sweep_body,
        grid=(NB,),
        in_specs=[
            pl.BlockSpec((R, C), lambda i: (i, z())),
            pl.BlockSpec((1, 1, R), lambda i: (i, z(), z())),
            pl.BlockSpec((R, C), lambda i: (z(), z())),
            pl.BlockSpec((C, R), lambda i: (z(), z())),
            pl.BlockSpec((8, R), lambda i: (z(), z())),
        ],
        out_specs=pl.BlockSpec((3 * C, C), lambda i: (z(), z())),
        out_shape=jax.ShapeDtypeStruct((3 * C, C), jnp.float32),
    )(yhat, y3d, ii, vi, w)


def _fold_body(a_ref, t_ref, o_ref):
    tbl = t_ref[0, :]
    vi = lax.broadcasted_iota(jnp.int32, (C, C), 0)
    t = (vi == tbl[None, :]).astype(jnp.float32)
    s = (jnp.float32(float(R)) * a_ref[0:C, :]
         + jnp.float32(128.0) * a_ref[C:2 * C, :]
         + a_ref[2 * C:3 * C, :])
    idx_sum = jnp.sum(t * s)
    loss = jnp.float32(1.0) - idx_sum * jnp.float32(1.0 / B)
    o_ref[...] = jnp.broadcast_to(loss, (1, 1))


def _fold_call(accs, table2d):
    return pl.pallas_call(
        _fold_body,
        out_shape=jax.ShapeDtypeStruct((1, 1), jnp.float32),
    )(accs, table2d)


def kernel(yhat, y):
    y32 = y.astype(jnp.int32)
    pres = _build_sc_presence()(y32)
    table = _build_sc_table()(pres)
    accs = _sweep_call(yhat, y32.reshape(NB, 1, R))
    loss = _fold_call(accs, table.reshape(1, C))
    return loss[0, 0]
